# Initial kernel scaffold; baseline (speedup 1.0000x reference)
import jax
import jax.numpy as jnp
from jax.experimental import pallas as pl


def kernel(curr_agent_info, curr_object_pos, demo_agent_info, demo_object_pos, actions, Wg1, Wg2, Wa, Ws, rho_layers, phi_w, psi_w, Wh1, Wh2):
    raise NotImplementedError("write your pallas kernel here")


# speedup vs baseline: 7.3469x; 7.3469x over previous
import functools

import jax
import jax.numpy as jnp
import numpy as np
from jax.experimental import pallas as pl

NUM_FREQS = 32
D = 128
HEADS = 8
DH = D // HEADS

_FREQS = (2.0 ** np.linspace(0.0, 8.0, NUM_FREQS)).astype(np.float32).reshape(1, NUM_FREQS)
_PERM = np.array([4 * f + c for c in range(4) for f in range(NUM_FREQS)], dtype=np.int32)
_HEADMASK = np.zeros((HEADS, D), dtype=np.float32)
for _h in range(HEADS):
    _HEADMASK[_h, _h * DH:(_h + 1) * DH] = 1.0
_F0 = np.concatenate([np.zeros((1, 2 * NUM_FREQS), np.float32),
                      np.ones((1, 2 * NUM_FREQS), np.float32)], axis=1)


def _fourier_cols(rx, ry, freqs):
    ax = rx * freqs
    ay = ry * freqs
    return jnp.concatenate([jnp.sin(ax), jnp.sin(ay), jnp.cos(ax), jnp.cos(ay)], axis=1)


def _reverse_kernel(act_ref, cop_ref, cai_ref, pos_ref, grip_ref, *, T):
    acts = act_ref[0]
    posx = cop_ref[0][:, 0:1]
    posy = cop_ref[0][:, 1:2]
    grip = cai_ref[0][:, 4:5]
    for t in range(T):
        dx = acts[t:t + 1, 0:1]
        dy = acts[t:t + 1, 1:2]
        dth = acts[t:t + 1, 2:3]
        sa = jnp.clip(acts[t:t + 1, 3:4], 0.0, 1.0)
        deltax = posx - dx
        deltay = posy - dy
        c = jnp.cos(-dth)
        s = jnp.sin(-dth)
        posx = c * deltax + s * deltay
        posy = -s * deltax + c * deltay
        pos_ref[0, t, :, 0:1] = posx
        pos_ref[0, t, :, 1:2] = posy
        change = jnp.round(sa) != jnp.round(grip)
        grip = jnp.where(change, jnp.broadcast_to(sa, grip.shape), grip)
        grip_ref[0, t, :, :] = grip


def _rho_kernel(ai_ref, sp_ref, wg1_ref, wg2_ref, wa_ref, ws_ref, rl_ref,
                hm_ref, hmt_ref, fr_ref, out_ref, *, A, M):
    ai = ai_ref[0]
    sp = sp_ref[0]
    hm = hm_ref[...]
    hmt = hmt_ref[...]
    freqs = fr_ref[...]

    F = jnp.tanh(sp @ wg1_ref[...]) @ wg2_ref[...]
    x_s = F @ ws_ref[...]
    x_a = ai @ wa_ref[...]

    spx = sp[:, 0:1]
    spy = sp[:, 1:2]
    ea_list = []
    for a in range(A):
        rx = ai[a:a + 1, 0:1] - spx
        ry = ai[a:a + 1, 1:2] - spy
        ea_list.append(_fourier_cols(rx, ry, freqs))

    for l in range(2):
        Wq = rl_ref[l, 0]
        Wk = rl_ref[l, 1]
        Wv = rl_ref[l, 2]
        We = rl_ref[l, 3]
        Wo = rl_ref[l, 4]
        q = x_a @ Wq
        k = x_s @ Wk
        v = x_s @ Wv
        rows = []
        for a in range(A):
            e = ea_list[a] @ We
            k_e = k + e
            v_e = v + e
            prod = q[a:a + 1, :] * k_e
            logits = (prod @ hmt) * 0.25
            mx = jnp.max(logits, axis=0, keepdims=True)
            ex = jnp.exp(logits - mx)
            den = jnp.sum(ex, axis=0, keepdims=True)
            alpha = ex / (den + 1e-9)
            af = alpha @ hm
            rows.append(jnp.sum(af * v_e, axis=0, keepdims=True))
        agg = jnp.concatenate(rows, axis=0)
        x_a = jnp.tanh(x_a + agg @ Wo)
    out_ref[0] = x_a


def _head_kernel(crho_ref, cai_ref, drho_ref, dpos_ref, prho_ref,
                 phiw_ref, psiw_ref, wh1_ref, wh2_ref,
                 hm_ref, hmt_ref, fr_ref, f0_ref, avg_ref, out_ref,
                 *, A, NL, T):
    x = crho_ref[0]
    cai = cai_ref[0]
    drho = drho_ref[0]
    dpos = dpos_ref[0]
    prho = prho_ref[0]
    hm = hm_ref[...]
    hmt = hmt_ref[...]
    freqs = fr_ref[...]

    qp = x @ phiw_ref[0]
    kall = drho @ phiw_ref[1]
    vall = drho @ phiw_ref[2]
    rows = []
    for a in range(A):
        sl = slice(a * NL, (a + 1) * NL)
        rx = cai[a:a + 1, 0:1] - dpos[sl, 0:1]
        ry = cai[a:a + 1, 1:2] - dpos[sl, 1:2]
        ea = _fourier_cols(rx, ry, freqs)
        e = ea @ phiw_ref[3]
        k_e = kall[sl, :] + e
        v_e = vall[sl, :] + e
        prod = qp[a:a + 1, :] * k_e
        logits = (prod @ hmt) * 0.25
        mx = jnp.max(logits, axis=0, keepdims=True)
        ex = jnp.exp(logits - mx)
        den = jnp.sum(ex, axis=0, keepdims=True)
        alpha = ex / (den + 1e-9)
        af = alpha @ hm
        rows.append(jnp.sum(af * v_e, axis=0, keepdims=True))
    ctx = jnp.tanh(x + jnp.concatenate(rows, axis=0) @ phiw_ref[4])

    e0 = f0_ref[...] @ psiw_ref[3]
    v = ctx @ psiw_ref[2] + e0
    alpha_c = 1.0 / (1.0 + 1e-9)
    w = (alpha_c * v) @ psiw_ref[4]
    wt = jnp.broadcast_to(w[None, :, :], (T, w.shape[0], w.shape[1]))
    wt = wt.reshape(T * w.shape[0], w.shape[1])
    final = jnp.tanh(prho + wt)

    h1 = jnp.tanh(final @ wh1_ref[...])
    h2 = h1 @ wh2_ref[...]
    out_ref[0] = avg_ref[...] @ h2


def _full(shape):
    nd = len(shape)
    return pl.BlockSpec(shape, lambda g: (0,) * nd)


def _lead(shape):
    nd = len(shape)
    return pl.BlockSpec((1,) + shape[1:], lambda g: (g,) + (0,) * (nd - 1))


def kernel(curr_agent_info, curr_object_pos, demo_agent_info, demo_object_pos,
           actions, Wg1, Wg2, Wa, Ws, rho_layers, phi_w, psi_w, Wh1, Wh2):
    B, A, _ = curr_agent_info.shape
    M = curr_object_pos.shape[1]
    _, N, L = demo_agent_info.shape[:3]
    T = actions.shape[1]
    NL = N * L
    f32 = jnp.float32

    freqs = jnp.asarray(_FREQS)
    hm = jnp.asarray(_HEADMASK)
    hmt = hm.T
    f0 = jnp.asarray(_F0)
    avg = jnp.kron(jnp.eye(T, dtype=f32), jnp.full((1, A), 1.0 / A, f32))

    rho_w = rho_layers.at[:, 3].set(rho_layers[:, 3][:, _PERM, :])
    phi_wp = phi_w.at[3].set(phi_w[3][_PERM, :])
    psi_wp = psi_w.at[3].set(psi_w[3][_PERM, :])

    pred_pos, grip = pl.pallas_call(
        functools.partial(_reverse_kernel, T=T),
        grid=(B,),
        in_specs=[_lead((B, T, 4)), _lead((B, M, 2)), _lead((B, A, 9))],
        out_specs=[_lead((B, T, M, 2)), _lead((B, T, A, 1))],
        out_shape=[jax.ShapeDtypeStruct((B, T, M, 2), f32),
                   jax.ShapeDtypeStruct((B, T, A, 1), f32)],
    )(actions, curr_object_pos, curr_agent_info)

    base4 = jnp.broadcast_to(curr_agent_info[:, None, :, 0:4], (B, T, A, 4))
    gate = jnp.broadcast_to(curr_agent_info[:, None, :, 5:6], (B, T, A, 1))
    pred_agent = jnp.concatenate([base4, grip, gate], axis=-1)
    pred_ai9 = jnp.concatenate(
        [pred_agent, jnp.zeros((B, T, A, 3), f32)], axis=-1)

    Gd = B * NL
    Gp = B * T
    Gtot = Gd + B + Gp
    ai_all = jnp.concatenate([
        demo_agent_info.reshape(Gd, A, 9),
        curr_agent_info,
        pred_ai9.reshape(Gp, A, 9),
    ], axis=0)
    sp_all = jnp.concatenate([
        demo_object_pos.reshape(Gd, M, 2),
        curr_object_pos,
        pred_pos.reshape(Gp, M, 2),
    ], axis=0)

    x_all = pl.pallas_call(
        functools.partial(_rho_kernel, A=A, M=M),
        grid=(Gtot,),
        in_specs=[_lead((Gtot, A, 9)), _lead((Gtot, M, 2)),
                  _full(Wg1.shape), _full(Wg2.shape), _full(Wa.shape),
                  _full(Ws.shape), _full(rho_w.shape),
                  _full(hm.shape), _full(hmt.shape), _full(freqs.shape)],
        out_specs=_lead((Gtot, A, D)),
        out_shape=jax.ShapeDtypeStruct((Gtot, A, D), f32),
    )(ai_all, sp_all, Wg1, Wg2, Wa, Ws, rho_w, hm, hmt, freqs)

    demo_rho_p = (x_all[:Gd].reshape(B, NL, A, D)
                  .transpose(0, 2, 1, 3).reshape(B, A * NL, D))
    curr_rho = x_all[Gd:Gd + B]
    pred_rho = x_all[Gd + B:].reshape(B, T * A, D)
    dpos_cols = (demo_agent_info[..., 0:2].reshape(B, NL, A, 2)
                 .transpose(0, 2, 1, 3).reshape(B, A * NL, 2))

    out = pl.pallas_call(
        functools.partial(_head_kernel, A=A, NL=NL, T=T),
        grid=(B,),
        in_specs=[_lead((B, A, D)), _lead((B, A, 9)),
                  _lead((B, A * NL, D)), _lead((B, A * NL, 2)),
                  _lead((B, T * A, D)),
                  _full(phi_wp.shape), _full(psi_wp.shape),
                  _full(Wh1.shape), _full(Wh2.shape),
                  _full(hm.shape), _full(hmt.shape), _full(freqs.shape),
                  _full(f0.shape), _full(avg.shape)],
        out_specs=_lead((B, T, 4)),
        out_shape=jax.ShapeDtypeStruct((B, T, 4), f32),
    )(curr_rho, curr_agent_info, demo_rho_p, dpos_cols, pred_rho,
      phi_wp, psi_wp, Wh1, Wh2, hm, hmt, freqs, f0, avg)

    return out


# speedup vs baseline: 9.7926x; 1.3329x over previous
import functools

import jax
import jax.numpy as jnp
import numpy as np
from jax.experimental import pallas as pl

NUM_FREQS = 32
D = 128
HEADS = 8
DH = D // HEADS

_FREQS = (2.0 ** np.linspace(0.0, 8.0, NUM_FREQS)).astype(np.float32).reshape(1, NUM_FREQS)
_PERM = np.array([4 * f + c for c in range(4) for f in range(NUM_FREQS)], dtype=np.int32)
_HEADMASK = np.zeros((HEADS, D), dtype=np.float32)
for _h in range(HEADS):
    _HEADMASK[_h, _h * DH:(_h + 1) * DH] = 1.0
_F0 = np.concatenate([np.zeros((1, 2 * NUM_FREQS), np.float32),
                      np.ones((1, 2 * NUM_FREQS), np.float32)], axis=1)


def _fourier_cols(rx, ry, freqs):
    ax = rx * freqs
    ay = ry * freqs
    return jnp.concatenate([jnp.sin(ax), jnp.sin(ay), jnp.cos(ax), jnp.cos(ay)], axis=1)


def _reverse_kernel(act_ref, cop_ref, cai_ref, pos_ref, grip_ref, *, T):
    acts = act_ref[0]
    posx = cop_ref[0][:, 0:1]
    posy = cop_ref[0][:, 1:2]
    grip = cai_ref[0][:, 4:5]
    for t in range(T):
        dx = acts[t:t + 1, 0:1]
        dy = acts[t:t + 1, 1:2]
        dth = acts[t:t + 1, 2:3]
        sa = jnp.clip(acts[t:t + 1, 3:4], 0.0, 1.0)
        deltax = posx - dx
        deltay = posy - dy
        c = jnp.cos(-dth)
        s = jnp.sin(-dth)
        posx = c * deltax + s * deltay
        posy = -s * deltax + c * deltay
        pos_ref[0, t, :, 0:1] = posx
        pos_ref[0, t, :, 1:2] = posy
        change = jnp.round(sa) != jnp.round(grip)
        grip = jnp.where(change, jnp.broadcast_to(sa, grip.shape), grip)
        grip_ref[0, t, :, :] = grip


def _rho_kernel(ai_ref, sp_ref, wg1_ref, wg2_ref, wa_ref, ws_ref, rl_ref,
                hm_ref, hmt_ref, fr_ref, out_ref, *, GB, A, M):
    GA = GB * A
    E = GA * M
    hm = hm_ref[...]
    hmt = hmt_ref[...]
    freqs = fr_ref[...]

    aif = ai_ref[...].reshape(GA, 9)
    spf = sp_ref[...].reshape(GB * M, 2)
    F = jnp.tanh(spf @ wg1_ref[...]) @ wg2_ref[...]
    x_s = F @ ws_ref[...]
    x_a = aif @ wa_ref[...]

    rx_parts = []
    ry_parts = []
    for g in range(GB):
        spx = sp_ref[g][:, 0:1]
        spy = sp_ref[g][:, 1:2]
        ai_g = ai_ref[g]
        for a in range(A):
            rx_parts.append(ai_g[a:a + 1, 0:1] - spx)
            ry_parts.append(ai_g[a:a + 1, 1:2] - spy)
    rx = jnp.concatenate(rx_parts, axis=0)
    ry = jnp.concatenate(ry_parts, axis=0)
    ea = _fourier_cols(rx, ry, freqs)

    for l in range(2):
        Wq = rl_ref[l, 0]
        Wk = rl_ref[l, 1]
        Wv = rl_ref[l, 2]
        We = rl_ref[l, 3]
        Wo = rl_ref[l, 4]
        q = x_a @ Wq
        k = x_s @ Wk
        v = x_s @ Wv
        e = ea @ We
        k4 = jnp.broadcast_to(k.reshape(GB, 1, M, D), (GB, A, M, D))
        v4 = jnp.broadcast_to(v.reshape(GB, 1, M, D), (GB, A, M, D))
        k_e = k4.reshape(E, D) + e
        v_e = v4.reshape(E, D) + e
        q_e = jnp.broadcast_to(q[:, None, :], (GA, M, D)).reshape(E, D)
        logits = ((q_e * k_e) @ hmt) * 0.25
        l3 = logits.reshape(GA, M, HEADS)
        mx = jnp.max(l3, axis=1, keepdims=True)
        ex = jnp.exp(l3 - mx)
        den = jnp.sum(ex, axis=1, keepdims=True)
        alpha = (ex / (den + 1e-9)).reshape(E, HEADS)
        af = alpha @ hm
        agg = jnp.sum((af * v_e).reshape(GA, M, D), axis=1)
        x_a = jnp.tanh(x_a + agg @ Wo)
    out_ref[...] = x_a.reshape(GB, A, D)


def _head_kernel(crho_ref, cai_ref, drho_ref, dpos_ref, prho_ref,
                 phiw_ref, psiw_ref, wh1_ref, wh2_ref,
                 hm_ref, hmt_ref, fr_ref, f0_ref, avg_ref, out_ref,
                 *, A, NL, T):
    x = crho_ref[0]
    cai = cai_ref[0]
    drho = drho_ref[0]
    dpos = dpos_ref[0]
    prho = prho_ref[0]
    hm = hm_ref[...]
    hmt = hmt_ref[...]
    freqs = fr_ref[...]

    qp = x @ phiw_ref[0]
    kall = drho @ phiw_ref[1]
    vall = drho @ phiw_ref[2]
    rows = []
    for a in range(A):
        sl = slice(a * NL, (a + 1) * NL)
        rx = cai[a:a + 1, 0:1] - dpos[sl, 0:1]
        ry = cai[a:a + 1, 1:2] - dpos[sl, 1:2]
        ea = _fourier_cols(rx, ry, freqs)
        e = ea @ phiw_ref[3]
        k_e = kall[sl, :] + e
        v_e = vall[sl, :] + e
        prod = qp[a:a + 1, :] * k_e
        logits = (prod @ hmt) * 0.25
        mx = jnp.max(logits, axis=0, keepdims=True)
        ex = jnp.exp(logits - mx)
        den = jnp.sum(ex, axis=0, keepdims=True)
        alpha = ex / (den + 1e-9)
        af = alpha @ hm
        rows.append(jnp.sum(af * v_e, axis=0, keepdims=True))
    ctx = jnp.tanh(x + jnp.concatenate(rows, axis=0) @ phiw_ref[4])

    e0 = f0_ref[...] @ psiw_ref[3]
    v = ctx @ psiw_ref[2] + e0
    alpha_c = 1.0 / (1.0 + 1e-9)
    w = (alpha_c * v) @ psiw_ref[4]
    wt = jnp.broadcast_to(w[None, :, :], (T, w.shape[0], w.shape[1]))
    wt = wt.reshape(T * w.shape[0], w.shape[1])
    final = jnp.tanh(prho + wt)

    h1 = jnp.tanh(final @ wh1_ref[...])
    h2 = h1 @ wh2_ref[...]
    out_ref[0] = avg_ref[...] @ h2


def _full(shape):
    nd = len(shape)
    return pl.BlockSpec(shape, lambda g: (0,) * nd)


def _lead(shape):
    nd = len(shape)
    return pl.BlockSpec((1,) + shape[1:], lambda g: (g,) + (0,) * (nd - 1))


def kernel(curr_agent_info, curr_object_pos, demo_agent_info, demo_object_pos,
           actions, Wg1, Wg2, Wa, Ws, rho_layers, phi_w, psi_w, Wh1, Wh2):
    B, A, _ = curr_agent_info.shape
    M = curr_object_pos.shape[1]
    _, N, L = demo_agent_info.shape[:3]
    T = actions.shape[1]
    NL = N * L
    f32 = jnp.float32

    freqs = jnp.asarray(_FREQS)
    hm = jnp.asarray(_HEADMASK)
    hmt = hm.T
    f0 = jnp.asarray(_F0)
    avg = jnp.kron(jnp.eye(T, dtype=f32), jnp.full((1, A), 1.0 / A, f32))

    rho_w = rho_layers.at[:, 3].set(rho_layers[:, 3][:, _PERM, :])
    phi_wp = phi_w.at[3].set(phi_w[3][_PERM, :])
    psi_wp = psi_w.at[3].set(psi_w[3][_PERM, :])

    pred_pos, grip = pl.pallas_call(
        functools.partial(_reverse_kernel, T=T),
        grid=(B,),
        in_specs=[_lead((B, T, 4)), _lead((B, M, 2)), _lead((B, A, 9))],
        out_specs=[_lead((B, T, M, 2)), _lead((B, T, A, 1))],
        out_shape=[jax.ShapeDtypeStruct((B, T, M, 2), f32),
                   jax.ShapeDtypeStruct((B, T, A, 1), f32)],
    )(actions, curr_object_pos, curr_agent_info)

    base4 = jnp.broadcast_to(curr_agent_info[:, None, :, 0:4], (B, T, A, 4))
    gate = jnp.broadcast_to(curr_agent_info[:, None, :, 5:6], (B, T, A, 1))
    pred_agent = jnp.concatenate([base4, grip, gate], axis=-1)
    pred_ai9 = jnp.concatenate(
        [pred_agent, jnp.zeros((B, T, A, 3), f32)], axis=-1)

    Gd = B * NL
    Gp = B * T
    Gtot = Gd + B + Gp
    ai_all = jnp.concatenate([
        demo_agent_info.reshape(Gd, A, 9),
        curr_agent_info,
        pred_ai9.reshape(Gp, A, 9),
    ], axis=0)
    sp_all = jnp.concatenate([
        demo_object_pos.reshape(Gd, M, 2),
        curr_object_pos,
        pred_pos.reshape(Gp, M, 2),
    ], axis=0)

    GB = 6
    assert Gtot % GB == 0
    x_all = pl.pallas_call(
        functools.partial(_rho_kernel, GB=GB, A=A, M=M),
        grid=(Gtot // GB,),
        in_specs=[pl.BlockSpec((GB, A, 9), lambda g: (g, 0, 0)),
                  pl.BlockSpec((GB, M, 2), lambda g: (g, 0, 0)),
                  _full(Wg1.shape), _full(Wg2.shape), _full(Wa.shape),
                  _full(Ws.shape), _full(rho_w.shape),
                  _full(hm.shape), _full(hmt.shape), _full(freqs.shape)],
        out_specs=pl.BlockSpec((GB, A, D), lambda g: (g, 0, 0)),
        out_shape=jax.ShapeDtypeStruct((Gtot, A, D), f32),
    )(ai_all, sp_all, Wg1, Wg2, Wa, Ws, rho_w, hm, hmt, freqs)

    demo_rho_p = (x_all[:Gd].reshape(B, NL, A, D)
                  .transpose(0, 2, 1, 3).reshape(B, A * NL, D))
    curr_rho = x_all[Gd:Gd + B]
    pred_rho = x_all[Gd + B:].reshape(B, T * A, D)
    dpos_cols = (demo_agent_info[..., 0:2].reshape(B, NL, A, 2)
                 .transpose(0, 2, 1, 3).reshape(B, A * NL, 2))

    out = pl.pallas_call(
        functools.partial(_head_kernel, A=A, NL=NL, T=T),
        grid=(B,),
        in_specs=[_lead((B, A, D)), _lead((B, A, 9)),
                  _lead((B, A * NL, D)), _lead((B, A * NL, 2)),
                  _lead((B, T * A, D)),
                  _full(phi_wp.shape), _full(psi_wp.shape),
                  _full(Wh1.shape), _full(Wh2.shape),
                  _full(hm.shape), _full(hmt.shape), _full(freqs.shape),
                  _full(f0.shape), _full(avg.shape)],
        out_specs=_lead((B, T, 4)),
        out_shape=jax.ShapeDtypeStruct((B, T, 4), f32),
    )(curr_rho, curr_agent_info, demo_rho_p, dpos_cols, pred_rho,
      phi_wp, psi_wp, Wh1, Wh2, hm, hmt, freqs, f0, avg)

    return out


# speedup vs baseline: 15.8910x; 1.6228x over previous
import functools

import jax
import jax.numpy as jnp
import numpy as np
from jax.experimental import pallas as pl

NUM_FREQS = 32
D = 128
HEADS = 8
DH = D // HEADS

_FREQS = (2.0 ** np.linspace(0.0, 8.0, NUM_FREQS)).astype(np.float32).reshape(1, NUM_FREQS)
_PERM = np.array([4 * f + c for c in range(4) for f in range(NUM_FREQS)], dtype=np.int32)
_HEADMASK = np.zeros((HEADS, D), dtype=np.float32)
for _h in range(HEADS):
    _HEADMASK[_h, _h * DH:(_h + 1) * DH] = 1.0
_F0 = np.concatenate([np.zeros((1, 2 * NUM_FREQS), np.float32),
                      np.ones((1, 2 * NUM_FREQS), np.float32)], axis=1)


_HALF_PI = np.float32(np.pi / 2)


def _fourier_cols(rx, ry, freqs):
    ax = rx * freqs
    ay = ry * freqs
    ang = jnp.concatenate([ax, ay, ax + _HALF_PI, ay + _HALF_PI], axis=1)
    return jnp.sin(ang)


def _reverse_kernel(act_ref, cop_ref, cai_ref, pos_ref, grip_ref, *, T):
    acts = act_ref[0]
    posx = cop_ref[0][:, 0:1]
    posy = cop_ref[0][:, 1:2]
    grip = cai_ref[0][:, 4:5]
    for t in range(T):
        dx = acts[t:t + 1, 0:1]
        dy = acts[t:t + 1, 1:2]
        dth = acts[t:t + 1, 2:3]
        sa = jnp.clip(acts[t:t + 1, 3:4], 0.0, 1.0)
        deltax = posx - dx
        deltay = posy - dy
        c = jnp.cos(-dth)
        s = jnp.sin(-dth)
        posx = c * deltax + s * deltay
        posy = -s * deltax + c * deltay
        pos_ref[0, t, :, 0:1] = posx
        pos_ref[0, t, :, 1:2] = posy
        change = jnp.round(sa) != jnp.round(grip)
        grip = jnp.where(change, jnp.broadcast_to(sa, grip.shape), grip)
        grip_ref[0, t, :, :] = grip


def _rho_kernel(ai_ref, sp_ref, wg1_ref, wg2_ref, wa_ref, ws_ref, rl_ref,
                hm_ref, hmt_ref, fr_ref, out_ref, *, GB, A, M):
    GA = GB * A
    E = GA * M
    hm = hm_ref[...]
    hmt = hmt_ref[...]
    freqs = fr_ref[...]

    aif = ai_ref[...].reshape(GA, 9)
    spf = sp_ref[...].reshape(GB * M, 2)
    F = jnp.tanh(spf @ wg1_ref[...]) @ wg2_ref[...]
    x_s = F @ ws_ref[...]
    x_a = aif @ wa_ref[...]

    rx_parts = []
    ry_parts = []
    for g in range(GB):
        spx = sp_ref[g][:, 0:1]
        spy = sp_ref[g][:, 1:2]
        ai_g = ai_ref[g]
        for a in range(A):
            rx_parts.append(ai_g[a:a + 1, 0:1] - spx)
            ry_parts.append(ai_g[a:a + 1, 1:2] - spy)
    rx = jnp.concatenate(rx_parts, axis=0)
    ry = jnp.concatenate(ry_parts, axis=0)
    ea = _fourier_cols(rx, ry, freqs)

    for l in range(2):
        Wq = rl_ref[l, 0]
        Wk = rl_ref[l, 1]
        Wv = rl_ref[l, 2]
        We = rl_ref[l, 3]
        Wo = rl_ref[l, 4]
        q = x_a @ Wq
        k = x_s @ Wk
        v = x_s @ Wv
        e = ea @ We
        k4 = jnp.broadcast_to(k.reshape(GB, 1, M, D), (GB, A, M, D))
        v4 = jnp.broadcast_to(v.reshape(GB, 1, M, D), (GB, A, M, D))
        k_e = k4.reshape(E, D) + e
        v_e = v4.reshape(E, D) + e
        q_e = jnp.broadcast_to(q[:, None, :], (GA, M, D)).reshape(E, D)
        logits = ((q_e * k_e) @ hmt) * 0.25
        l3 = logits.reshape(GA, M, HEADS)
        mx = jnp.max(l3, axis=1, keepdims=True)
        ex = jnp.exp(l3 - mx)
        den = jnp.sum(ex, axis=1, keepdims=True)
        alpha = (ex / (den + 1e-9)).reshape(E, HEADS)
        af = alpha @ hm
        agg = jnp.sum((af * v_e).reshape(GA, M, D), axis=1)
        x_a = jnp.tanh(x_a + agg @ Wo)
    out_ref[...] = x_a.reshape(GB, A, D)


def _head_kernel(crho_ref, cai_ref, drho_ref, dpos_ref, prho_ref,
                 phiw_ref, psiw_ref, wh1_ref, wh2_ref,
                 hm_ref, hmt_ref, fr_ref, f0_ref, avg_ref, out_ref,
                 *, A, NL, T):
    x = crho_ref[0]
    cai = cai_ref[0]
    drho = drho_ref[0]
    dpos = dpos_ref[0]
    prho = prho_ref[0]
    hm = hm_ref[...]
    hmt = hmt_ref[...]
    freqs = fr_ref[...]

    qp = x @ phiw_ref[0]
    kall = drho @ phiw_ref[1]
    vall = drho @ phiw_ref[2]
    rows = []
    for a in range(A):
        sl = slice(a * NL, (a + 1) * NL)
        rx = cai[a:a + 1, 0:1] - dpos[sl, 0:1]
        ry = cai[a:a + 1, 1:2] - dpos[sl, 1:2]
        ea = _fourier_cols(rx, ry, freqs)
        e = ea @ phiw_ref[3]
        k_e = kall[sl, :] + e
        v_e = vall[sl, :] + e
        prod = qp[a:a + 1, :] * k_e
        logits = (prod @ hmt) * 0.25
        mx = jnp.max(logits, axis=0, keepdims=True)
        ex = jnp.exp(logits - mx)
        den = jnp.sum(ex, axis=0, keepdims=True)
        alpha = ex / (den + 1e-9)
        af = alpha @ hm
        rows.append(jnp.sum(af * v_e, axis=0, keepdims=True))
    ctx = jnp.tanh(x + jnp.concatenate(rows, axis=0) @ phiw_ref[4])

    e0 = f0_ref[...] @ psiw_ref[3]
    v = ctx @ psiw_ref[2] + e0
    alpha_c = 1.0 / (1.0 + 1e-9)
    w = (alpha_c * v) @ psiw_ref[4]
    wt = jnp.broadcast_to(w[None, :, :], (T, w.shape[0], w.shape[1]))
    wt = wt.reshape(T * w.shape[0], w.shape[1])
    final = jnp.tanh(prho + wt)

    h1 = jnp.tanh(final @ wh1_ref[...])
    h2 = h1 @ wh2_ref[...]
    out_ref[0] = avg_ref[...] @ h2


def _full(shape):
    nd = len(shape)
    return pl.BlockSpec(shape, lambda g: (0,) * nd)


def _lead(shape):
    nd = len(shape)
    return pl.BlockSpec((1,) + shape[1:], lambda g: (g,) + (0,) * (nd - 1))


def kernel(curr_agent_info, curr_object_pos, demo_agent_info, demo_object_pos,
           actions, Wg1, Wg2, Wa, Ws, rho_layers, phi_w, psi_w, Wh1, Wh2):
    B, A, _ = curr_agent_info.shape
    M = curr_object_pos.shape[1]
    _, N, L = demo_agent_info.shape[:3]
    T = actions.shape[1]
    NL = N * L
    f32 = jnp.float32

    freqs = jnp.asarray(_FREQS)
    hm = jnp.asarray(_HEADMASK)
    hmt = hm.T
    f0 = jnp.asarray(_F0)
    avg = jnp.kron(jnp.eye(T, dtype=f32), jnp.full((1, A), 1.0 / A, f32))

    rho_w = rho_layers.at[:, 3].set(rho_layers[:, 3][:, _PERM, :])
    phi_wp = phi_w.at[3].set(phi_w[3][_PERM, :])
    psi_wp = psi_w.at[3].set(psi_w[3][_PERM, :])

    pred_pos, grip = pl.pallas_call(
        functools.partial(_reverse_kernel, T=T),
        grid=(B,),
        in_specs=[_lead((B, T, 4)), _lead((B, M, 2)), _lead((B, A, 9))],
        out_specs=[_lead((B, T, M, 2)), _lead((B, T, A, 1))],
        out_shape=[jax.ShapeDtypeStruct((B, T, M, 2), f32),
                   jax.ShapeDtypeStruct((B, T, A, 1), f32)],
    )(actions, curr_object_pos, curr_agent_info)

    base4 = jnp.broadcast_to(curr_agent_info[:, None, :, 0:4], (B, T, A, 4))
    gate = jnp.broadcast_to(curr_agent_info[:, None, :, 5:6], (B, T, A, 1))
    pred_agent = jnp.concatenate([base4, grip, gate], axis=-1)
    pred_ai9 = jnp.concatenate(
        [pred_agent, jnp.zeros((B, T, A, 3), f32)], axis=-1)

    Gd = B * NL
    Gp = B * T
    Gtot = Gd + B + Gp
    ai_all = jnp.concatenate([
        demo_agent_info.reshape(Gd, A, 9),
        curr_agent_info,
        pred_ai9.reshape(Gp, A, 9),
    ], axis=0)
    sp_all = jnp.concatenate([
        demo_object_pos.reshape(Gd, M, 2),
        curr_object_pos,
        pred_pos.reshape(Gp, M, 2),
    ], axis=0)

    GB = 6
    assert Gtot % GB == 0
    x_all = pl.pallas_call(
        functools.partial(_rho_kernel, GB=GB, A=A, M=M),
        grid=(Gtot // GB,),
        in_specs=[pl.BlockSpec((GB, A, 9), lambda g: (g, 0, 0)),
                  pl.BlockSpec((GB, M, 2), lambda g: (g, 0, 0)),
                  _full(Wg1.shape), _full(Wg2.shape), _full(Wa.shape),
                  _full(Ws.shape), _full(rho_w.shape),
                  _full(hm.shape), _full(hmt.shape), _full(freqs.shape)],
        out_specs=pl.BlockSpec((GB, A, D), lambda g: (g, 0, 0)),
        out_shape=jax.ShapeDtypeStruct((Gtot, A, D), f32),
    )(ai_all, sp_all, Wg1, Wg2, Wa, Ws, rho_w, hm, hmt, freqs)

    demo_rho_p = (x_all[:Gd].reshape(B, NL, A, D)
                  .transpose(0, 2, 1, 3).reshape(B, A * NL, D))
    curr_rho = x_all[Gd:Gd + B]
    pred_rho = x_all[Gd + B:].reshape(B, T * A, D)
    dpos_cols = (demo_agent_info[..., 0:2].reshape(B, NL, A, 2)
                 .transpose(0, 2, 1, 3).reshape(B, A * NL, 2))

    out = pl.pallas_call(
        functools.partial(_head_kernel, A=A, NL=NL, T=T),
        grid=(B,),
        in_specs=[_lead((B, A, D)), _lead((B, A, 9)),
                  _lead((B, A * NL, D)), _lead((B, A * NL, 2)),
                  _lead((B, T * A, D)),
                  _full(phi_wp.shape), _full(psi_wp.shape),
                  _full(Wh1.shape), _full(Wh2.shape),
                  _full(hm.shape), _full(hmt.shape), _full(freqs.shape),
                  _full(f0.shape), _full(avg.shape)],
        out_specs=_lead((B, T, 4)),
        out_shape=jax.ShapeDtypeStruct((B, T, 4), f32),
    )(curr_rho, curr_agent_info, demo_rho_p, dpos_cols, pred_rho,
      phi_wp, psi_wp, Wh1, Wh2, hm, hmt, freqs, f0, avg)

    return out


# speedup vs baseline: 25.3471x; 1.5951x over previous
import functools

import jax
import jax.numpy as jnp
import numpy as np
from jax.experimental import pallas as pl

NUM_FREQS = 32
D = 128
HEADS = 8
DH = D // HEADS

_FREQS = (2.0 ** np.linspace(0.0, 8.0, NUM_FREQS)).astype(np.float32).reshape(1, NUM_FREQS)
_PERM = np.array([4 * f + c for c in range(4) for f in range(NUM_FREQS)], dtype=np.int32)
_HEADMASK = np.zeros((HEADS, D), dtype=np.float32)
for _h in range(HEADS):
    _HEADMASK[_h, _h * DH:(_h + 1) * DH] = 1.0
_F0 = np.concatenate([np.zeros((1, 2 * NUM_FREQS), np.float32),
                      np.ones((1, 2 * NUM_FREQS), np.float32)], axis=1)


_HALF_PI = np.float32(np.pi / 2)


def _fourier_cols(rx, ry, freqs):
    ax = rx * freqs
    ay = ry * freqs
    ang = jnp.concatenate([ax, ay, ax + _HALF_PI, ay + _HALF_PI], axis=1)
    return jnp.sin(ang)


def _reverse_kernel(act_ref, cop_ref, cai_ref, pos_ref, grip_ref, *, T):
    acts = act_ref[0]
    posx = cop_ref[0][:, 0:1]
    posy = cop_ref[0][:, 1:2]
    grip = cai_ref[0][:, 4:5]
    for t in range(T):
        dx = acts[t:t + 1, 0:1]
        dy = acts[t:t + 1, 1:2]
        dth = acts[t:t + 1, 2:3]
        sa = jnp.clip(acts[t:t + 1, 3:4], 0.0, 1.0)
        deltax = posx - dx
        deltay = posy - dy
        c = jnp.cos(-dth)
        s = jnp.sin(-dth)
        posx = c * deltax + s * deltay
        posy = -s * deltax + c * deltay
        pos_ref[0, t, :, 0:1] = posx
        pos_ref[0, t, :, 1:2] = posy
        change = jnp.round(sa) != jnp.round(grip)
        grip = jnp.where(change, jnp.broadcast_to(sa, grip.shape), grip)
        grip_ref[0, t, :, :] = grip


def _rho_kernel(ai_ref, sp_ref, wg1_ref, wg2_ref, wa_ref, ws_ref, rl_ref,
                hm_ref, hmt_ref, fr_ref, out_ref, *, GB, A, M):
    GA = GB * A
    E = GA * M
    hm = hm_ref[...]
    hmt = hmt_ref[...]
    freqs = fr_ref[...]

    aif = ai_ref[...].reshape(GA, 9)
    spf = sp_ref[...].reshape(GB * M, 2)
    F = jnp.tanh(spf @ wg1_ref[...]) @ wg2_ref[...]
    x_s = F @ ws_ref[...]
    x_a = aif @ wa_ref[...]

    H = D // 2
    sx_parts = [sp_ref[g][:, 0:1] for g in range(GB)]
    sy_parts = [sp_ref[g][:, 1:2] for g in range(GB)]
    sx = jnp.concatenate(sx_parts, axis=0)
    sy = jnp.concatenate(sy_parts, axis=0)
    scs = _fourier_cols(sx, sy, freqs)
    aca = _fourier_cols(aif[:, 0:1], aif[:, 1:2], freqs)
    a_s = aca[:, 0:H]
    a_c = aca[:, H:D]
    as_e = jnp.broadcast_to(a_s[:, None, :], (GA, M, H)).reshape(E, H)
    ac_e = jnp.broadcast_to(a_c[:, None, :], (GA, M, H)).reshape(E, H)
    scs4 = scs.reshape(GB, 1, M, D)
    ss_e = jnp.broadcast_to(scs4[..., 0:H], (GB, A, M, H)).reshape(E, H)
    sc_e = jnp.broadcast_to(scs4[..., H:D], (GB, A, M, H)).reshape(E, H)
    ea = jnp.concatenate([as_e * sc_e - ac_e * ss_e,
                          ac_e * sc_e + as_e * ss_e], axis=1)

    for l in range(2):
        Wq = rl_ref[l, 0]
        Wk = rl_ref[l, 1]
        Wv = rl_ref[l, 2]
        We = rl_ref[l, 3]
        Wo = rl_ref[l, 4]
        q = x_a @ Wq
        k = x_s @ Wk
        v = x_s @ Wv
        e = ea @ We
        k4 = jnp.broadcast_to(k.reshape(GB, 1, M, D), (GB, A, M, D))
        v4 = jnp.broadcast_to(v.reshape(GB, 1, M, D), (GB, A, M, D))
        k_e = k4.reshape(E, D) + e
        v_e = v4.reshape(E, D) + e
        q_e = jnp.broadcast_to(q[:, None, :], (GA, M, D)).reshape(E, D)
        logits = ((q_e * k_e) @ hmt) * 0.25
        l3 = logits.reshape(GA, M, HEADS)
        mx = jnp.max(l3, axis=1, keepdims=True)
        ex = jnp.exp(l3 - mx)
        den = jnp.sum(ex, axis=1, keepdims=True)
        alpha = (ex / (den + 1e-9)).reshape(E, HEADS)
        af = alpha @ hm
        agg = jnp.sum((af * v_e).reshape(GA, M, D), axis=1)
        x_a = jnp.tanh(x_a + agg @ Wo)
    out_ref[...] = x_a.reshape(GB, A, D)


def _head_kernel(crho_ref, cai_ref, drho_ref, dpos_ref, prho_ref,
                 phiw_ref, psiw_ref, wh1_ref, wh2_ref,
                 hm_ref, hmt_ref, fr_ref, f0_ref, avg_ref, out_ref,
                 *, A, NL, T):
    x = crho_ref[0]
    cai = cai_ref[0]
    drho = drho_ref[0]
    dpos = dpos_ref[0]
    prho = prho_ref[0]
    hm = hm_ref[...]
    hmt = hmt_ref[...]
    freqs = fr_ref[...]

    qp = x @ phiw_ref[0]
    kall = drho @ phiw_ref[1]
    vall = drho @ phiw_ref[2]
    rows = []
    for a in range(A):
        sl = slice(a * NL, (a + 1) * NL)
        rx = cai[a:a + 1, 0:1] - dpos[sl, 0:1]
        ry = cai[a:a + 1, 1:2] - dpos[sl, 1:2]
        ea = _fourier_cols(rx, ry, freqs)
        e = ea @ phiw_ref[3]
        k_e = kall[sl, :] + e
        v_e = vall[sl, :] + e
        prod = qp[a:a + 1, :] * k_e
        logits = (prod @ hmt) * 0.25
        mx = jnp.max(logits, axis=0, keepdims=True)
        ex = jnp.exp(logits - mx)
        den = jnp.sum(ex, axis=0, keepdims=True)
        alpha = ex / (den + 1e-9)
        af = alpha @ hm
        rows.append(jnp.sum(af * v_e, axis=0, keepdims=True))
    ctx = jnp.tanh(x + jnp.concatenate(rows, axis=0) @ phiw_ref[4])

    e0 = f0_ref[...] @ psiw_ref[3]
    v = ctx @ psiw_ref[2] + e0
    alpha_c = 1.0 / (1.0 + 1e-9)
    w = (alpha_c * v) @ psiw_ref[4]
    wt = jnp.broadcast_to(w[None, :, :], (T, w.shape[0], w.shape[1]))
    wt = wt.reshape(T * w.shape[0], w.shape[1])
    final = jnp.tanh(prho + wt)

    h1 = jnp.tanh(final @ wh1_ref[...])
    h2 = h1 @ wh2_ref[...]
    out_ref[0] = avg_ref[...] @ h2


def _full(shape):
    nd = len(shape)
    return pl.BlockSpec(shape, lambda g: (0,) * nd)


def _lead(shape):
    nd = len(shape)
    return pl.BlockSpec((1,) + shape[1:], lambda g: (g,) + (0,) * (nd - 1))


def kernel(curr_agent_info, curr_object_pos, demo_agent_info, demo_object_pos,
           actions, Wg1, Wg2, Wa, Ws, rho_layers, phi_w, psi_w, Wh1, Wh2):
    B, A, _ = curr_agent_info.shape
    M = curr_object_pos.shape[1]
    _, N, L = demo_agent_info.shape[:3]
    T = actions.shape[1]
    NL = N * L
    f32 = jnp.float32

    freqs = jnp.asarray(_FREQS)
    hm = jnp.asarray(_HEADMASK)
    hmt = hm.T
    f0 = jnp.asarray(_F0)
    avg = jnp.kron(jnp.eye(T, dtype=f32), jnp.full((1, A), 1.0 / A, f32))

    rho_w = rho_layers.at[:, 3].set(rho_layers[:, 3][:, _PERM, :])
    phi_wp = phi_w.at[3].set(phi_w[3][_PERM, :])
    psi_wp = psi_w.at[3].set(psi_w[3][_PERM, :])

    pred_pos, grip = pl.pallas_call(
        functools.partial(_reverse_kernel, T=T),
        grid=(B,),
        in_specs=[_lead((B, T, 4)), _lead((B, M, 2)), _lead((B, A, 9))],
        out_specs=[_lead((B, T, M, 2)), _lead((B, T, A, 1))],
        out_shape=[jax.ShapeDtypeStruct((B, T, M, 2), f32),
                   jax.ShapeDtypeStruct((B, T, A, 1), f32)],
    )(actions, curr_object_pos, curr_agent_info)

    base4 = jnp.broadcast_to(curr_agent_info[:, None, :, 0:4], (B, T, A, 4))
    gate = jnp.broadcast_to(curr_agent_info[:, None, :, 5:6], (B, T, A, 1))
    pred_agent = jnp.concatenate([base4, grip, gate], axis=-1)
    pred_ai9 = jnp.concatenate(
        [pred_agent, jnp.zeros((B, T, A, 3), f32)], axis=-1)

    Gd = B * NL
    Gp = B * T
    Gtot = Gd + B + Gp
    ai_all = jnp.concatenate([
        demo_agent_info.reshape(Gd, A, 9),
        curr_agent_info,
        pred_ai9.reshape(Gp, A, 9),
    ], axis=0)
    sp_all = jnp.concatenate([
        demo_object_pos.reshape(Gd, M, 2),
        curr_object_pos,
        pred_pos.reshape(Gp, M, 2),
    ], axis=0)

    GB = 6
    assert Gtot % GB == 0
    x_all = pl.pallas_call(
        functools.partial(_rho_kernel, GB=GB, A=A, M=M),
        grid=(Gtot // GB,),
        in_specs=[pl.BlockSpec((GB, A, 9), lambda g: (g, 0, 0)),
                  pl.BlockSpec((GB, M, 2), lambda g: (g, 0, 0)),
                  _full(Wg1.shape), _full(Wg2.shape), _full(Wa.shape),
                  _full(Ws.shape), _full(rho_w.shape),
                  _full(hm.shape), _full(hmt.shape), _full(freqs.shape)],
        out_specs=pl.BlockSpec((GB, A, D), lambda g: (g, 0, 0)),
        out_shape=jax.ShapeDtypeStruct((Gtot, A, D), f32),
    )(ai_all, sp_all, Wg1, Wg2, Wa, Ws, rho_w, hm, hmt, freqs)

    demo_rho_p = (x_all[:Gd].reshape(B, NL, A, D)
                  .transpose(0, 2, 1, 3).reshape(B, A * NL, D))
    curr_rho = x_all[Gd:Gd + B]
    pred_rho = x_all[Gd + B:].reshape(B, T * A, D)
    dpos_cols = (demo_agent_info[..., 0:2].reshape(B, NL, A, 2)
                 .transpose(0, 2, 1, 3).reshape(B, A * NL, 2))

    out = pl.pallas_call(
        functools.partial(_head_kernel, A=A, NL=NL, T=T),
        grid=(B,),
        in_specs=[_lead((B, A, D)), _lead((B, A, 9)),
                  _lead((B, A * NL, D)), _lead((B, A * NL, 2)),
                  _lead((B, T * A, D)),
                  _full(phi_wp.shape), _full(psi_wp.shape),
                  _full(Wh1.shape), _full(Wh2.shape),
                  _full(hm.shape), _full(hmt.shape), _full(freqs.shape),
                  _full(f0.shape), _full(avg.shape)],
        out_specs=_lead((B, T, 4)),
        out_shape=jax.ShapeDtypeStruct((B, T, 4), f32),
    )(curr_rho, curr_agent_info, demo_rho_p, dpos_cols, pred_rho,
      phi_wp, psi_wp, Wh1, Wh2, hm, hmt, freqs, f0, avg)

    return out
